# baseline (device time: 201369 ns/iter reference)
import jax
import jax.numpy as jnp
from jax import lax
from jax.experimental import pallas as pl
from jax.experimental.pallas import tpu as pltpu


def kernel(Q, K, V):
    b, sq, h, d = Q.shape
    skv = K.shape[1]
    scale = d ** -0.5

    def partial_body(q_ref, k_ref, v_ref, o_ref, m_ref, l_ref):
        for hi in range(h):
            lanes = slice(hi * d, (hi + 1) * d)
            q = q_ref[0, :, lanes]
            k = k_ref[0, :, lanes]
            v = v_ref[0, :, lanes]
            s = lax.dot_general(
                q, k, (((1,), (1,)), ((), ())),
                preferred_element_type=jnp.float32,
            ) * scale
            m = jnp.max(s, axis=1, keepdims=True)
            p = jnp.exp(s - m)
            l = jnp.sum(p, axis=1, keepdims=True)
            o = lax.dot_general(
                p, v, (((1,), (0,)), ((), ())),
                preferred_element_type=jnp.float32,
            )
            o_ref[0, :, lanes] = o
            m_ref[0, :, hi:hi + 1] = m
            l_ref[0, :, hi:hi + 1] = l

    o_p, m_p, l_p = pl.pallas_call(
        partial_body,
        grid=(b,),
        in_specs=[
            pl.BlockSpec((1, sq, h * d), lambda bi: (bi, 0, 0)),
            pl.BlockSpec((1, skv, h * d), lambda bi: (bi, 0, 0)),
            pl.BlockSpec((1, skv, h * d), lambda bi: (bi, 0, 0)),
        ],
        out_specs=[
            pl.BlockSpec((1, sq, h * d), lambda bi: (bi, 0, 0)),
            pl.BlockSpec((1, sq, h), lambda bi: (bi, 0, 0)),
            pl.BlockSpec((1, sq, h), lambda bi: (bi, 0, 0)),
        ],
        out_shape=[
            jax.ShapeDtypeStruct((b, sq, h * d), jnp.float32),
            jax.ShapeDtypeStruct((b, sq, h), jnp.float32),
            jax.ShapeDtypeStruct((b, sq, h), jnp.float32),
        ],
        compiler_params=pltpu.CompilerParams(
            vmem_limit_bytes=48 * 1024 * 1024,
        ),
    )(
        Q.reshape(b, sq, h * d),
        K.reshape(b, skv, h * d),
        V.reshape(b, skv, h * d),
    )
    o_p = o_p.reshape(b, sq, h, d)

    def combine_body(o_ref, m_ref, l_ref, out_ref,
                     o_rx, m_rx, l_rx, send_sems, recv_sems):
        my_x = lax.axis_index("x")
        my_y = lax.axis_index("y")
        my_z = lax.axis_index("z")
        partner = (my_x, 1 - my_y, my_z)

        barrier = pltpu.get_barrier_semaphore()
        pl.semaphore_signal(barrier, inc=1, device_id=partner,
                            device_id_type=pl.DeviceIdType.MESH)
        pl.semaphore_wait(barrier, 1)

        copies = []
        for i, (src, dst) in enumerate(
            ((o_ref, o_rx), (m_ref, m_rx), (l_ref, l_rx))
        ):
            c = pltpu.make_async_remote_copy(
                src_ref=src, dst_ref=dst,
                send_sem=send_sems.at[i], recv_sem=recv_sems.at[i],
                device_id=partner, device_id_type=pl.DeviceIdType.MESH,
            )
            c.start()
            copies.append(c)
        for c in copies:
            c.wait()

        m_a = m_ref[...]
        m_b = m_rx[...]
        m_n = jnp.maximum(m_a, m_b)
        ca = jnp.exp(m_a - m_n)
        cb = jnp.exp(m_b - m_n)
        l_n = ca * l_ref[...] + cb * l_rx[...]
        wa = (ca / l_n)[..., None]
        wb = (cb / l_n)[..., None]
        out_ref[...] = o_ref[...] * wa + o_rx[...] * wb

    return pl.pallas_call(
        combine_body,
        in_specs=[pl.BlockSpec(memory_space=pltpu.VMEM)] * 3,
        out_specs=pl.BlockSpec(memory_space=pltpu.VMEM),
        out_shape=jax.ShapeDtypeStruct((b, sq, h, d), jnp.float32),
        scratch_shapes=[
            pltpu.VMEM((b, sq, h, d), jnp.float32),
            pltpu.VMEM((b, sq, h), jnp.float32),
            pltpu.VMEM((b, sq, h), jnp.float32),
            pltpu.SemaphoreType.DMA((3,)),
            pltpu.SemaphoreType.DMA((3,)),
        ],
        compiler_params=pltpu.CompilerParams(collective_id=0),
    )(o_p, m_p, l_p)


# device time: 75905 ns/iter; 2.6529x vs baseline; 2.6529x over previous
import jax
import jax.numpy as jnp
from jax import lax
from jax.experimental import pallas as pl
from jax.experimental.pallas import tpu as pltpu


def partial_call(Q, K, V):
    b, sq, h, d = Q.shape
    skv = K.shape[1]
    scale = d ** -0.5
    F = h * sq

    def partial_body(q_ref, k_ref, v_ref, o_ref, st_ref):
        qf = q_ref[0]
        fI = lax.broadcasted_iota(jnp.int32, (F, F), 0)
        rI = lax.broadcasted_iota(jnp.int32, (F, F), 1)
        perm_in = jnp.where((fI % sq) * h + fI // sq == rI, 1.0, 0.0)
        qperm = lax.dot_general(
            perm_in, qf, (((1,), (0,)), ((), ())),
            preferred_element_type=jnp.float32,
        )
        k2 = k_ref[0]
        s = lax.dot_general(
            k2, qperm, (((1,), (1,)), ((), ())),
            preferred_element_type=jnp.float32,
        ) * scale
        s3 = s.reshape(skv, h, F)
        hk = lax.broadcasted_iota(jnp.int32, (h, F), 0)
        hq = lax.broadcasted_iota(jnp.int32, (h, F), 1) // sq
        mask = hk == hq
        mrow = jnp.max(s3, axis=0)
        mvec = jnp.sum(jnp.where(mask, mrow, 0.0), axis=0, keepdims=True)
        e = jnp.exp(s3 - mvec.reshape(1, 1, F))
        p3 = jnp.where(mask.reshape(1, h, F), e, 0.0)
        lrow = jnp.sum(p3, axis=0)
        lvec = jnp.sum(lrow, axis=0, keepdims=True)
        p2 = p3.reshape(skv * h, F)
        o_all = lax.dot_general(
            p2, v_ref[0], (((0,), (0,)), ((), ())),
            preferred_element_type=jnp.float32,
        )
        o_ref[0] = o_all
        rowI = lax.broadcasted_iota(jnp.int32, (8, F), 0)
        st_ref[0] = jnp.where(rowI == 0, mvec, lvec)

    o_p, st_p = pl.pallas_call(
        partial_body,
        grid=(b,),
        in_specs=[
            pl.BlockSpec((1, sq * h, d), lambda bi: (bi, 0, 0)),
            pl.BlockSpec((1, skv * h, d), lambda bi: (bi, 0, 0)),
            pl.BlockSpec((1, skv * h, d), lambda bi: (bi, 0, 0)),
        ],
        out_specs=[
            pl.BlockSpec((1, F, d), lambda bi: (bi, 0, 0)),
            pl.BlockSpec((1, 8, F), lambda bi: (bi, 0, 0)),
        ],
        out_shape=[
            jax.ShapeDtypeStruct((b, F, d), jnp.float32),
            jax.ShapeDtypeStruct((b, 8, F), jnp.float32),
        ],
        compiler_params=pltpu.CompilerParams(
            vmem_limit_bytes=100 * 1024 * 1024,
        ),
    )(
        Q.reshape(b, sq * h, d),
        K.reshape(b, skv * h, d),
        V.reshape(b, skv * h, d),
    )
    return o_p, st_p


def kernel(Q, K, V):
    b, sq, h, d = Q.shape
    F = h * sq
    o_p, st_p = partial_call(Q, K, V)

    def combine_body(o_ref, st_ref, out_ref,
                     o_rx, st_rx, send_sems, recv_sems):
        my_x = lax.axis_index("x")
        my_y = lax.axis_index("y")
        my_z = lax.axis_index("z")
        partner = (my_x, 1 - my_y, my_z)

        barrier = pltpu.get_barrier_semaphore()
        pl.semaphore_signal(barrier, inc=1, device_id=partner,
                            device_id_type=pl.DeviceIdType.MESH)
        pl.semaphore_wait(barrier, 1)

        copies = []
        for i, (src, dst) in enumerate(
            ((o_ref, o_rx), (st_ref, st_rx))
        ):
            c = pltpu.make_async_remote_copy(
                src_ref=src, dst_ref=dst,
                send_sem=send_sems.at[i], recv_sem=recv_sems.at[i],
                device_id=partner, device_id_type=pl.DeviceIdType.MESH,
            )
            c.start()
            copies.append(c)
        for c in copies:
            c.wait()

        m_a = st_ref[:, 0:1, :]
        l_a = st_ref[:, 1:2, :]
        m_b = st_rx[:, 0:1, :]
        l_b = st_rx[:, 1:2, :]
        m_n = jnp.maximum(m_a, m_b)
        ca = jnp.exp(m_a - m_n)
        cb = jnp.exp(m_b - m_n)
        l_n = ca * l_a + cb * l_b
        wa = ca / l_n
        wb = cb / l_n

        fI = lax.broadcasted_iota(jnp.int32, (F, F), 0)
        gI = lax.broadcasted_iota(jnp.int32, (F, F), 1)
        eye = fI == gI
        perm_out = jnp.where((fI % h) * sq + fI // h == gI, 1.0, 0.0)
        for bi in range(b):
            da = jnp.where(eye, jnp.broadcast_to(wa[bi], (F, F)), 0.0)
            db = jnp.where(eye, jnp.broadcast_to(wb[bi], (F, F)), 0.0)
            comb = lax.dot_general(
                da, o_ref[bi], (((1,), (0,)), ((), ())),
                preferred_element_type=jnp.float32,
            ) + lax.dot_general(
                db, o_rx[bi], (((1,), (0,)), ((), ())),
                preferred_element_type=jnp.float32,
            )
            out_ref[bi] = lax.dot_general(
                perm_out, comb, (((1,), (0,)), ((), ())),
                preferred_element_type=jnp.float32,
            )

    out = pl.pallas_call(
        combine_body,
        in_specs=[pl.BlockSpec(memory_space=pltpu.VMEM)] * 2,
        out_specs=pl.BlockSpec(memory_space=pltpu.VMEM),
        out_shape=jax.ShapeDtypeStruct((b, F, d), jnp.float32),
        scratch_shapes=[
            pltpu.VMEM((b, F, d), jnp.float32),
            pltpu.VMEM((b, 8, F), jnp.float32),
            pltpu.SemaphoreType.DMA((2,)),
            pltpu.SemaphoreType.DMA((2,)),
        ],
        compiler_params=pltpu.CompilerParams(collective_id=0),
    )(o_p, st_p)
    return out.reshape(b, sq, h, d)
